# baseline (speedup 1.0000x reference)
import functools

import jax
import jax.numpy as jnp
from jax import lax
from jax.experimental import pallas as pl
from jax.experimental.pallas import tpu as pltpu
from jax.experimental.pallas import tpu_sc as plsc

R = 128
C = 32768
L = 16
NC = 2
NS = 16
NW = NC * NS
ROWS_PER_W = R // NW
STEPS = C // L

_mesh = plsc.VectorSubcoreMesh(core_axis_name="c", subcore_axis_name="s")


def _shuffle(x, idx):
    return lax.gather(
        x, idx[:, None],
        lax.GatherDimensionNumbers(
            offset_dims=(), collapsed_slice_dims=(0,), start_index_map=(0,)),
        slice_sizes=(1,),
        mode=lax.GatherScatterMode.PROMISE_IN_BOUNDS)


@functools.partial(
    pl.kernel,
    out_type=jax.ShapeDtypeStruct((R, C), jnp.float32),
    mesh=_mesh,
    scratch_types=[
        pltpu.VMEM((C,), jnp.float32),
        pltpu.VMEM((C,), jnp.float32),
    ],
    compiler_params=pltpu.CompilerParams(needs_layout_passes=False),
)
def _argmax_onehot(data_hbm, out_hbm, in_v, out_v):
    wid = lax.axis_index("s") * NC + lax.axis_index("c")
    lanes = lax.iota(jnp.int32, L)
    zeros = jnp.zeros((L,), jnp.float32)
    ones = jnp.ones((L,), jnp.float32)

    for r in range(ROWS_PER_W):
        row = wid * ROWS_PER_W + r
        pltpu.sync_copy(data_hbm.at[row], in_v)

        def step(j, carry, _r=r):
            bv, bi = carry
            v = in_v[pl.ds(j * L, L)]
            out_v[pl.ds(j * L, L)] = zeros
            idx = j * L + lanes
            upd = v > bv
            bv = jnp.where(upd, v, bv)
            bi = jnp.where(upd, idx, bi)
            return bv, bi

        init = (jnp.full((L,), -jnp.inf, jnp.float32),
                jnp.zeros((L,), jnp.int32))
        bv, bi = lax.fori_loop(0, STEPS, step, init)

        for k in (8, 4, 2, 1):
            pv = _shuffle(bv, lanes ^ k)
            pi = _shuffle(bi, lanes ^ k)
            take = (pv > bv) | ((pv == bv) & (pi < bi))
            bv = jnp.where(take, pv, bv)
            bi = jnp.where(take, pi, bi)

        plsc.store_scatter(out_v, [bi], ones, mask=lanes == 0)
        pltpu.sync_copy(out_v, out_hbm.at[row])


def kernel(data):
    return _argmax_onehot(data)


# speedup vs baseline: 1.5861x; 1.5861x over previous
import functools

import jax
import jax.numpy as jnp
from jax import lax
from jax.experimental import pallas as pl
from jax.experimental.pallas import tpu as pltpu
from jax.experimental.pallas import tpu_sc as plsc

R = 128
C = 32768
L = 16
NC = 2
NS = 16
NW = NC * NS
ROWS_PER_W = R // NW
STEPS = C // L

_mesh = plsc.VectorSubcoreMesh(core_axis_name="c", subcore_axis_name="s")


def _shuffle(x, idx):
    return lax.gather(
        x, idx[:, None],
        lax.GatherDimensionNumbers(
            offset_dims=(), collapsed_slice_dims=(0,), start_index_map=(0,)),
        slice_sizes=(1,),
        mode=lax.GatherScatterMode.PROMISE_IN_BOUNDS)


@functools.partial(
    pl.kernel,
    out_type=jax.ShapeDtypeStruct((R, C), jnp.float32),
    mesh=_mesh,
    scratch_types=[
        pltpu.VMEM((C,), jnp.float32),
        pltpu.VMEM((C,), jnp.float32),
    ],
    compiler_params=pltpu.CompilerParams(needs_layout_passes=False),
)
def _argmax_onehot(data_hbm, out_hbm, in_v, out_v):
    wid = lax.axis_index("s") * NC + lax.axis_index("c")
    lanes = lax.iota(jnp.int32, L)
    zeros = jnp.zeros((L,), jnp.float32)
    ones = jnp.ones((L,), jnp.float32)

    U = 8

    def zfill(t, _):
        base = t * (U * L)
        for k in range(U):
            out_v[pl.ds(base + k * L, L)] = zeros
        return 0

    lax.fori_loop(0, STEPS // U, zfill, 0)

    for r in range(ROWS_PER_W):
        row = wid * ROWS_PER_W + r
        pltpu.sync_copy(data_hbm.at[row], in_v)

        def step(t, carry):
            bv, bi = carry
            base = t * (U * L)
            for k in range(U):
                v = in_v[pl.ds(base + k * L, L)]
                idx = (base + k * L) + lanes
                upd = v > bv
                bv = jnp.where(upd, v, bv)
                bi = jnp.where(upd, idx, bi)
            return bv, bi

        init = (jnp.full((L,), -jnp.inf, jnp.float32),
                jnp.zeros((L,), jnp.int32))
        bv, bi = lax.fori_loop(0, STEPS // U, step, init)

        for k in (8, 4, 2, 1):
            pv = _shuffle(bv, lanes ^ k)
            pi = _shuffle(bi, lanes ^ k)
            take = (pv > bv) | ((pv == bv) & (pi < bi))
            bv = jnp.where(take, pv, bv)
            bi = jnp.where(take, pi, bi)

        plsc.store_scatter(out_v, [bi], ones, mask=lanes == 0)
        pltpu.sync_copy(out_v, out_hbm.at[row])
        plsc.store_scatter(out_v, [bi], zeros, mask=lanes == 0)


def kernel(data):
    return _argmax_onehot(data)
